# Initial kernel scaffold; baseline (speedup 1.0000x reference)
import jax
import jax.numpy as jnp
from jax.experimental import pallas as pl


def kernel(x, edge_index, edge_weight, W):
    raise NotImplementedError("write your pallas kernel here")


# speedup vs baseline: 4.3991x; 4.3991x over previous
import functools

import jax
import jax.numpy as jnp
from jax import lax
from jax.experimental import pallas as pl
from jax.experimental.pallas import tpu as pltpu
from jax.experimental.pallas import tpu_sc as plsc

_NC = 2
_NS = 16
_LANES = 16
_C = 128


def _lane_broadcast(vec, l):
    idx = jnp.full((_LANES, 1), l, jnp.int32)
    dn = lax.GatherDimensionNumbers(
        offset_dims=(), collapsed_slice_dims=(0,), start_index_map=(0,))
    return lax.gather(vec, idx, dn, slice_sizes=(1,),
                      mode=lax.GatherScatterMode.PROMISE_IN_BOUNDS)


def _sc_body(n_chunks, n_rows_tile, d, x_hbm, src_hbm, dst_hbm, w_hbm,
             zeros_hbm, out_hbm, src_v, dst_v, w_v, buf, acc, sem):
    c = lax.axis_index("c")
    s = lax.axis_index("s")

    pltpu.sync_copy(src_hbm.at[c, s], src_v)
    pltpu.sync_copy(dst_hbm.at[c, s], dst_v)
    pltpu.sync_copy(w_hbm.at[c, s], w_v)
    pltpu.sync_copy(zeros_hbm, acc.at[pl.ds(s * n_rows_tile, n_rows_tile)])
    plsc.subcore_barrier()

    def chunk_body(j, carry):
        pltpu.async_copy(x_hbm.at[src_v.at[j]], buf, sem).wait()

        def group_body(g, carry2):
            wvec = w_v[j, pl.ds(g * _LANES, _LANES)]
            for l in range(_LANES):
                wb = _lane_broadcast(wvec, l)
                e = g * _LANES + l
                for k in range(d // _LANES):
                    sl = pl.ds(k * _LANES, _LANES)
                    buf[e, sl] = buf[e, sl] * wb
            return carry2

        lax.fori_loop(0, _C // _LANES, group_body, 0)
        pltpu.sync_copy(buf, acc.at[dst_v.at[j]], add=True)
        return carry

    lax.fori_loop(0, n_chunks, chunk_body, 0)
    plsc.subcore_barrier()
    pltpu.sync_copy(acc.at[pl.ds(s * n_rows_tile, n_rows_tile)],
                    out_hbm.at[c, pl.ds(s * n_rows_tile, n_rows_tile)])


def _tc_body(p_ref, w_ref, o_ref):
    a = p_ref[0] + p_ref[1]
    o_ref[...] = jnp.maximum(
        jnp.dot(a, w_ref[...], preferred_element_type=jnp.float32), 0.0)


def kernel(x, edge_index, edge_weight, W):
    n, d = x.shape
    e = edge_index.shape[1]
    nw = _NC * _NS
    per_tile = -(-e // (nw * _C)) * _C
    n_chunks = per_tile // _C
    e_pad = nw * per_tile
    n_pad = -(-n // (_NS * 8)) * (_NS * 8)
    n_rows_tile = n_pad // _NS

    src = jnp.pad(edge_index[0], (0, e_pad - e)).reshape(_NC, _NS, n_chunks, _C)
    dst = jnp.pad(edge_index[1], (0, e_pad - e)).reshape(_NC, _NS, n_chunks, _C)
    w = jnp.pad(edge_weight, (0, e_pad - e)).reshape(_NC, _NS, n_chunks, _C)
    zeros = jnp.zeros((n_rows_tile, d), jnp.float32)

    mesh = plsc.VectorSubcoreMesh(core_axis_name="c", subcore_axis_name="s")
    sc = pl.kernel(
        functools.partial(_sc_body, n_chunks, n_rows_tile, d),
        out_type=jax.ShapeDtypeStruct((_NC, n_pad, d), jnp.float32),
        mesh=mesh,
        scratch_types=[
            pltpu.VMEM((n_chunks, _C), jnp.int32),
            pltpu.VMEM((n_chunks, _C), jnp.int32),
            pltpu.VMEM((n_chunks, _C), jnp.float32),
            pltpu.VMEM((_C, d), jnp.float32),
            pltpu.VMEM_SHARED((n_pad, d), jnp.float32),
            pltpu.SemaphoreType.DMA,
        ],
    )
    partials = sc(x, src, dst, w, zeros)

    bn = 1000
    out = pl.pallas_call(
        _tc_body,
        grid=(n // bn,),
        in_specs=[
            pl.BlockSpec((_NC, bn, d), lambda i: (0, i, 0)),
            pl.BlockSpec((d, d), lambda i: (0, 0)),
        ],
        out_specs=pl.BlockSpec((bn, d), lambda i: (i, 0)),
        out_shape=jax.ShapeDtypeStruct((n, d), jnp.float32),
    )(partials, W)
    return out
